# baseline (speedup 1.0000x reference)
import functools

import jax
import jax.numpy as jnp
from jax import lax
from jax.experimental import pallas as pl
from jax.experimental.pallas import tpu as pltpu
from jax.experimental.pallas import tpu_sc as plsc

_MAX_CTX = 32768
_B, _L, _D = 4, 8192, 1024

_NC, _NS = 2, 16
_NW = _NC * _NS
_SC_BASE = 24576
_SC_ROWS = 8192
_ROWS_PER_W = _SC_ROWS // _NW
_SC_NBUF = 4
_SC_CHUNK = 16
_SC_NCHUNK = _ROWS_PER_W // _SC_CHUNK
assert _SC_NCHUNK % _SC_NBUF == 0

_TC_ROWS = _SC_BASE
_TC_NBUF = 4
_TC_CHUNK = 2048
_TC_NCHUNK = _TC_ROWS // _TC_CHUNK
assert _TC_NCHUNK % _TC_NBUF == 0

_mesh = plsc.VectorSubcoreMesh(core_axis_name="c", subcore_axis_name="s")


@functools.partial(
    pl.kernel,
    out_type=jax.ShapeDtypeStruct((8,), jnp.float32),
    mesh=_mesh,
    scratch_types=[
        pltpu.VMEM((_SC_NBUF, _SC_CHUNK, _D), jnp.float32),
        pltpu.SemaphoreType.DMA((_SC_NBUF,)),
        pltpu.SemaphoreType.DMA((_SC_NBUF,)),
        pltpu.SemaphoreType.DMA((_SC_NBUF,)),
    ],
    compiler_params=pltpu.CompilerParams(has_side_effects=True),
)
def _sc_copy(src_hbm, h_hbm, mem_hbm, _dummy, buf, rsem, w1sem, w2sem):
    wid = lax.axis_index("s") * _NC + lax.axis_index("c")
    base = _SC_BASE + wid * _ROWS_PER_W

    ngroups = _SC_NCHUNK // _SC_NBUF
    for g in range(ngroups):
        for b in range(_SC_NBUF):
            c = base + (g * _SC_NBUF + b) * _SC_CHUNK
            if g > 0:
                pc = c - _SC_NBUF * _SC_CHUNK
                pltpu.make_async_copy(
                    buf.at[b], mem_hbm.at[pl.ds(pc, _SC_CHUNK)], w1sem.at[b]
                ).wait()
                pltpu.make_async_copy(
                    buf.at[b], h_hbm.at[3, pl.ds(pc - _SC_BASE, _SC_CHUNK)], w2sem.at[b]
                ).wait()
            pltpu.make_async_copy(
                src_hbm.at[pl.ds(c, _SC_CHUNK)], buf.at[b], rsem.at[b]
            ).start()
        for b in range(_SC_NBUF):
            c = base + (g * _SC_NBUF + b) * _SC_CHUNK
            pltpu.make_async_copy(
                src_hbm.at[pl.ds(c, _SC_CHUNK)], buf.at[b], rsem.at[b]
            ).wait()
            pltpu.make_async_copy(
                buf.at[b], mem_hbm.at[pl.ds(c, _SC_CHUNK)], w1sem.at[b]
            ).start()
            pltpu.make_async_copy(
                buf.at[b], h_hbm.at[3, pl.ds(c - _SC_BASE, _SC_CHUNK)], w2sem.at[b]
            ).start()
    for b in range(_SC_NBUF):
        c = base + ((_SC_NCHUNK - _SC_NBUF) + b) * _SC_CHUNK
        pltpu.make_async_copy(
            buf.at[b], mem_hbm.at[pl.ds(c, _SC_CHUNK)], w1sem.at[b]
        ).wait()
        pltpu.make_async_copy(
            buf.at[b], h_hbm.at[3, pl.ds(c - _SC_BASE, _SC_CHUNK)], w2sem.at[b]
        ).wait()


def _alloc_h_kernel(dst_ref):
    pass


def _alloc_mem_kernel(dst_ref):
    pass


def _tc_copy_kernel(src_ref, h_ref, mem_ref, _dummy, buf, rsem, w1sem, w2sem):
    chunks_per_batch = _L // _TC_CHUNK

    def h_slot(c):
        return (c // chunks_per_batch, pl.ds((c % chunks_per_batch) * _TC_CHUNK, _TC_CHUNK))

    ngroups = _TC_NCHUNK // _TC_NBUF
    for g in range(ngroups):
        for b in range(_TC_NBUF):
            c = g * _TC_NBUF + b
            if g > 0:
                pc = c - _TC_NBUF
                pltpu.make_async_copy(
                    buf.at[b], mem_ref.at[pl.ds(pc * _TC_CHUNK, _TC_CHUNK), :], w1sem.at[b]
                ).wait()
                bi, rs = h_slot(pc)
                pltpu.make_async_copy(
                    buf.at[b], h_ref.at[bi, rs, :], w2sem.at[b]
                ).wait()
            pltpu.make_async_copy(
                src_ref.at[pl.ds(c * _TC_CHUNK, _TC_CHUNK), :], buf.at[b], rsem.at[b]
            ).start()
        for b in range(_TC_NBUF):
            c = g * _TC_NBUF + b
            pltpu.make_async_copy(
                src_ref.at[pl.ds(c * _TC_CHUNK, _TC_CHUNK), :], buf.at[b], rsem.at[b]
            ).wait()
            pltpu.make_async_copy(
                buf.at[b], mem_ref.at[pl.ds(c * _TC_CHUNK, _TC_CHUNK), :], w1sem.at[b]
            ).start()
            bi, rs = h_slot(c)
            pltpu.make_async_copy(
                buf.at[b], h_ref.at[bi, rs, :], w2sem.at[b]
            ).start()
    g = ngroups - 1
    for b in range(_TC_NBUF):
        c = g * _TC_NBUF + b
        pltpu.make_async_copy(
            buf.at[b], mem_ref.at[pl.ds(c * _TC_CHUNK, _TC_CHUNK), :], w1sem.at[b]
        ).wait()
        bi, rs = h_slot(c)
        pltpu.make_async_copy(
            buf.at[b], h_ref.at[bi, rs, :], w2sem.at[b]
        ).wait()


def kernel(h, mem):
    b, l, d = h.shape
    assert (b, l, d) == (_B, _L, _D)
    flat = h.reshape(b * l, d)

    out_h = pl.pallas_call(
        _alloc_h_kernel,
        out_specs=pl.BlockSpec(memory_space=pl.ANY),
        out_shape=jax.ShapeDtypeStruct((b, l, d), h.dtype),
    )()
    out_mem = pl.pallas_call(
        _alloc_mem_kernel,
        out_specs=pl.BlockSpec(memory_space=pl.ANY),
        out_shape=jax.ShapeDtypeStruct((b * l, d), h.dtype),
    )()

    tc_dummy = pl.pallas_call(
        _tc_copy_kernel,
        in_specs=[
            pl.BlockSpec(memory_space=pl.ANY),
            pl.BlockSpec(memory_space=pl.ANY),
            pl.BlockSpec(memory_space=pl.ANY),
        ],
        out_specs=pl.BlockSpec(memory_space=pl.ANY),
        out_shape=jax.ShapeDtypeStruct((8, 128), h.dtype),
        scratch_shapes=[
            pltpu.VMEM((_TC_NBUF, _TC_CHUNK, _D), h.dtype),
            pltpu.SemaphoreType.DMA((_TC_NBUF,)),
            pltpu.SemaphoreType.DMA((_TC_NBUF,)),
            pltpu.SemaphoreType.DMA((_TC_NBUF,)),
        ],
        compiler_params=pltpu.CompilerParams(
            has_side_effects=True,
            disable_bounds_checks=True,
            disable_semaphore_checks=True,
            skip_device_barrier=True,
        ),
    )(flat, out_h, out_mem)

    sc_dummy = _sc_copy(flat, out_h, out_mem)

    out_h, out_mem, _, _ = lax.optimization_barrier(
        (out_h, out_mem, tc_dummy, sc_dummy)
    )
    return (out_h, out_mem)


# speedup vs baseline: 1.1528x; 1.1528x over previous
import jax
import jax.numpy as jnp
from jax.experimental import pallas as pl
from jax.experimental.pallas import tpu as pltpu

_MAX_CTX = 32768
_D = 1024
_NBUF = 4
_CHUNK = 2048
_NCHUNKS = _MAX_CTX // _CHUNK
assert _NCHUNKS % _NBUF == 0


def _copy_kernel(src_ref, out_h_ref, out_mem_ref, buf, rsem, w1sem, w2sem):
    rows_per_batch = out_h_ref.shape[1]
    chunks_per_batch = rows_per_batch // _CHUNK

    def h_slot(c):
        return (c // chunks_per_batch, pl.ds((c % chunks_per_batch) * _CHUNK, _CHUNK))

    ngroups = _NCHUNKS // _NBUF
    for g in range(ngroups):
        for b in range(_NBUF):
            c = g * _NBUF + b
            if g > 0:
                pc = c - _NBUF
                pltpu.make_async_copy(
                    buf.at[b], out_mem_ref.at[pl.ds(pc * _CHUNK, _CHUNK), :], w1sem.at[b]
                ).wait()
                bi, rs = h_slot(pc)
                pltpu.make_async_copy(
                    buf.at[b], out_h_ref.at[bi, rs, :], w2sem.at[b]
                ).wait()
            pltpu.make_async_copy(
                src_ref.at[pl.ds(c * _CHUNK, _CHUNK), :], buf.at[b], rsem.at[b]
            ).start()
        for b in range(_NBUF):
            c = g * _NBUF + b
            pltpu.make_async_copy(
                src_ref.at[pl.ds(c * _CHUNK, _CHUNK), :], buf.at[b], rsem.at[b]
            ).wait()
            pltpu.make_async_copy(
                buf.at[b], out_mem_ref.at[pl.ds(c * _CHUNK, _CHUNK), :], w1sem.at[b]
            ).start()
            bi, rs = h_slot(c)
            pltpu.make_async_copy(
                buf.at[b], out_h_ref.at[bi, rs, :], w2sem.at[b]
            ).start()
    g = ngroups - 1
    for b in range(_NBUF):
        c = g * _NBUF + b
        pltpu.make_async_copy(
            buf.at[b], out_mem_ref.at[pl.ds(c * _CHUNK, _CHUNK), :], w1sem.at[b]
        ).wait()
        bi, rs = h_slot(c)
        pltpu.make_async_copy(
            buf.at[b], out_h_ref.at[bi, rs, :], w2sem.at[b]
        ).wait()


def kernel(h, mem):
    b, l, d = h.shape
    assert b * l == _MAX_CTX and d == _D
    flat = h.reshape(b * l, d)
    out_h, new_mem = pl.pallas_call(
        _copy_kernel,
        in_specs=[pl.BlockSpec(memory_space=pl.ANY)],
        out_specs=[
            pl.BlockSpec(memory_space=pl.ANY),
            pl.BlockSpec(memory_space=pl.ANY),
        ],
        out_shape=[
            jax.ShapeDtypeStruct((b, l, d), h.dtype),
            jax.ShapeDtypeStruct((b * l, d), h.dtype),
        ],
        scratch_shapes=[
            pltpu.VMEM((_NBUF, _CHUNK, _D), h.dtype),
            pltpu.SemaphoreType.DMA((_NBUF,)),
            pltpu.SemaphoreType.DMA((_NBUF,)),
            pltpu.SemaphoreType.DMA((_NBUF,)),
        ],
        compiler_params=pltpu.CompilerParams(
            disable_bounds_checks=True,
            disable_semaphore_checks=True,
            skip_device_barrier=True,
        ),
    )(flat)
    return (out_h, new_mem)
